# baseline (speedup 1.0000x reference)
import jax
import jax.numpy as jnp
from jax.experimental import pallas as pl
from jax.experimental.pallas import tpu as pltpu

_BT = 128


def _mem_cell_kernel(x_ref, s_ref, keys_ref, uw_ref, vt_ref, b_ref, out_ref):
    x = x_ref[...]
    keys = keys_ref[...]
    j_slots, e = keys.shape
    uw = uw_ref[...]

    kvb = jax.lax.dot_general(keys, vt_ref[...], (((1,), (0,)), ((), ())),
                              preferred_element_type=jnp.float32) + b_ref[...]

    for j in range(j_slots):
        sl = slice(j * e, (j + 1) * e)
        s_j = s_ref[:, sl]
        z = jnp.sum(x * (s_j + keys[j:j + 1, :]), axis=-1, keepdims=True)
        g = 1.0 / (1.0 + jnp.exp(-z))
        su = jax.lax.dot_general(jnp.concatenate([s_j, x], axis=1), uw,
                                 (((1,), (0,)), ((), ())),
                                 preferred_element_type=jnp.float32)
        cand = jnp.maximum(su + kvb[j:j + 1, :], 0.0)
        s_next = s_j + g * cand
        ss = jnp.sum(s_next * s_next, axis=-1, keepdims=True)
        out_ref[:, sl] = s_next * jax.lax.rsqrt(jnp.maximum(ss, 1e-30))


def kernel(x, state, keys, U, V, W, bias):
    b, je = state.shape
    j, e = keys.shape
    uw = jnp.concatenate([U.T, W.T], axis=0)
    grid = (b // _BT,)
    return pl.pallas_call(
        _mem_cell_kernel,
        grid=grid,
        in_specs=[
            pl.BlockSpec((_BT, e), lambda i: (i, 0)),
            pl.BlockSpec((_BT, je), lambda i: (i, 0)),
            pl.BlockSpec((j, e), lambda i: (0, 0)),
            pl.BlockSpec((2 * e, e), lambda i: (0, 0)),
            pl.BlockSpec((e, e), lambda i: (0, 0)),
            pl.BlockSpec((1, e), lambda i: (0, 0)),
        ],
        out_specs=pl.BlockSpec((_BT, je), lambda i: (i, 0)),
        out_shape=jax.ShapeDtypeStruct((b, je), jnp.float32),
        compiler_params=pltpu.CompilerParams(
            dimension_semantics=("parallel",),
            vmem_limit_bytes=50 * 1024 * 1024,
        ),
    )(x, state, keys, uw, V.T, bias.reshape(1, e))


# speedup vs baseline: 1.0759x; 1.0759x over previous
import jax
import jax.numpy as jnp
from jax.experimental import pallas as pl
from jax.experimental.pallas import tpu as pltpu

_BT = 256


def _mem_cell_kernel(x_ref, s_ref, keys_ref, uw_ref, vt_ref, b_ref, out_ref):
    x = x_ref[...]
    keys = keys_ref[...]
    j_slots, e = keys.shape
    uw = uw_ref[...]

    kvb = jax.lax.dot_general(keys, vt_ref[...], (((1,), (0,)), ((), ())),
                              preferred_element_type=jnp.float32) + b_ref[...]

    for j in range(j_slots):
        sl = slice(j * e, (j + 1) * e)
        s_j = s_ref[:, sl]
        z = jnp.sum(x * (s_j + keys[j:j + 1, :]), axis=-1, keepdims=True)
        g = 1.0 / (1.0 + jnp.exp(-z))
        su = jax.lax.dot_general(jnp.concatenate([s_j, x], axis=1), uw,
                                 (((1,), (0,)), ((), ())),
                                 preferred_element_type=jnp.float32)
        cand = jnp.maximum(su + kvb[j:j + 1, :], 0.0)
        s_next = s_j + g * cand
        ss = jnp.sum(s_next * s_next, axis=-1, keepdims=True)
        out_ref[:, sl] = s_next * jax.lax.rsqrt(jnp.maximum(ss, 1e-30))


def kernel(x, state, keys, U, V, W, bias):
    b, je = state.shape
    j, e = keys.shape
    uw = jnp.concatenate([U.T, W.T], axis=0)
    grid = (b // _BT,)
    return pl.pallas_call(
        _mem_cell_kernel,
        grid=grid,
        in_specs=[
            pl.BlockSpec((_BT, e), lambda i: (i, 0)),
            pl.BlockSpec((_BT, je), lambda i: (i, 0)),
            pl.BlockSpec((j, e), lambda i: (0, 0)),
            pl.BlockSpec((2 * e, e), lambda i: (0, 0)),
            pl.BlockSpec((e, e), lambda i: (0, 0)),
            pl.BlockSpec((1, e), lambda i: (0, 0)),
        ],
        out_specs=pl.BlockSpec((_BT, je), lambda i: (i, 0)),
        out_shape=jax.ShapeDtypeStruct((b, je), jnp.float32),
        compiler_params=pltpu.CompilerParams(
            dimension_semantics=("parallel",),
            vmem_limit_bytes=50 * 1024 * 1024,
        ),
    )(x, state, keys, uw, V.T, bias.reshape(1, e))
